# baseline (speedup 1.0000x reference)
import functools

import jax
import jax.numpy as jnp
from jax import lax
from jax.experimental import pallas as pl
from jax.experimental.pallas import tpu as pltpu
from jax.experimental.pallas import tpu_sc as plsc

DIM = 256
BLK_MAJ = 8
BLK_R = 256
BUF_ROWS = 256
LANES = 16


N_BUF = 6


def _tc_body(ys_ref, pad_ref, w_ref, b_ref, y_hbm, scratch, sems):
    i = pl.program_id(0)
    j = pl.program_id(1)
    nj = pl.num_programs(1)
    step = i * nj + j
    n = pl.num_programs(0) * nj
    slot = lax.rem(step, N_BUF)

    def _dma(s, st):
        bi = st // nj
        bj = st - bi * nj
        return pltpu.make_async_copy(
            scratch.at[s],
            y_hbm.at[pl.ds(bi * BLK_MAJ, BLK_MAJ), pl.ds(bj * BLK_R, BLK_R)],
            sems.at[s],
        )

    @pl.when(step >= N_BUF)
    def _wait_prev():
        _dma(slot, step - N_BUF).wait()

    ys = ys_ref[...]
    pad = pad_ref[...]
    w = w_ref[0, :]
    b = b_ref[0, :]
    m = jnp.where(pad, 0.0, 1.0)
    scratch[slot] = (
        ys[:, :, None] * w[None, None, :] + b[None, None, :]
    ) * m[:, :, None]
    _dma(slot, step).start()

    @pl.when(step == n - 1)
    def _drain():
        for k in range(N_BUF):
            _dma(lax.rem(step - k, N_BUF), step - k).wait()


def _tc_y(y_support, padding, w2, b2):
    B, N = y_support.shape
    return pl.pallas_call(
        _tc_body,
        grid=(B // BLK_MAJ, N // BLK_R),
        in_specs=[
            pl.BlockSpec((BLK_MAJ, BLK_R), lambda i, j: (i, j)),
            pl.BlockSpec((BLK_MAJ, BLK_R), lambda i, j: (i, j)),
            pl.BlockSpec((1, DIM), lambda i, j: (0, 0)),
            pl.BlockSpec((1, DIM), lambda i, j: (0, 0)),
        ],
        out_specs=pl.BlockSpec(memory_space=pl.ANY),
        out_shape=jax.ShapeDtypeStruct((B, N, DIM), jnp.float32),
        scratch_shapes=[
            pltpu.VMEM((N_BUF, BLK_MAJ, BLK_R, DIM), jnp.float32),
            pltpu.SemaphoreType.DMA((N_BUF,)),
        ],
    )(y_support, padding, w2, b2)


def _sc_fill(mask_table, batch, n_query):
    info = plsc.get_sparse_core_info()
    nc, ns = info.num_cores, info.num_subcores
    nw = nc * ns
    total_elems = batch * n_query * DIM
    chunk_elems = BUF_ROWS * DIM
    elems_per_w = total_elems // nw
    n_chunk = elems_per_w // chunk_elems
    mesh = plsc.VectorSubcoreMesh(core_axis_name="c", subcore_axis_name="s")

    @functools.partial(
        pl.kernel,
        out_type=jax.ShapeDtypeStruct((total_elems,), jnp.float32),
        mesh=mesh,
        scratch_types=[
            pltpu.VMEM((DIM,), jnp.float32),
            pltpu.VMEM((chunk_elems,), jnp.float32),
            pltpu.SemaphoreType.DMA,
        ],
    )
    def yq_fill(mt_hbm, out_hbm, row_v, buf_v, sem):
        wid = lax.axis_index("s") * nc + lax.axis_index("c")
        base = wid * elems_per_w
        pltpu.sync_copy(mt_hbm, row_v)
        vs = [row_v[pl.ds(LANES * d, LANES)] for d in range(DIM // LANES)]

        def fill_body(i, carry):
            for d in range(DIM // LANES):
                buf_v[pl.ds(i * DIM + LANES * d, LANES)] = vs[d]
            return carry

        lax.fori_loop(0, BUF_ROWS, fill_body, 0)

        def fire_body(j, carry):
            pltpu.make_async_copy(
                buf_v, out_hbm.at[pl.ds(base + j * chunk_elems, chunk_elems)], sem
            ).start()
            return carry

        lax.fori_loop(0, n_chunk, fire_body, 0)

        def drain_body(j, carry):
            pltpu.make_async_copy(
                buf_v, out_hbm.at[pl.ds(base, chunk_elems)], sem
            ).wait()
            return carry

        lax.fori_loop(0, n_chunk, drain_body, 0)

    return yq_fill(mask_table.reshape(DIM))


def kernel(y_support, padding_obs_support, n_obs_query, W_y, b_y, mask_table):
    batch, n_sup = y_support.shape
    w2 = W_y.reshape(1, DIM)
    b2 = b_y.reshape(1, DIM)

    yq = _sc_fill(mask_table, batch, n_sup)
    y = _tc_y(y_support, padding_obs_support, w2, b2)

    return (
        y,
        yq.reshape(batch, n_sup, 1, DIM),
    )


# speedup vs baseline: 1.0335x; 1.0335x over previous
import functools

import jax
import jax.numpy as jnp
from jax import lax
from jax.experimental import pallas as pl
from jax.experimental.pallas import tpu as pltpu
from jax.experimental.pallas import tpu_sc as plsc

DIM = 256
BLK_MAJ = 8
BLK_R = 256
BUF_ROWS = 256
LANES = 16


N_BUF = 6


def _tc_body(ys_ref, m_ref, w_ref, b_ref, y_hbm, scratch, sems):
    i = pl.program_id(0)
    n = pl.num_programs(0)
    slot = lax.rem(i, N_BUF)

    def _dma(s, st):
        return pltpu.make_async_copy(
            scratch.at[s],
            y_hbm.at[pl.ds(st * BLK_MAJ, BLK_MAJ)],
            sems.at[s],
        )

    @pl.when(i >= N_BUF)
    def _wait_prev():
        _dma(slot, i - N_BUF).wait()

    ys = ys_ref[...]
    m = m_ref[...]
    w = w_ref[0, :]
    b = b_ref[0, :]
    scratch[slot] = (
        ys[:, :, None] * w[None, None, :] + b[None, None, :]
    ) * m[:, :, None]
    _dma(slot, i).start()

    @pl.when(i == n - 1)
    def _drain():
        for k in range(N_BUF):
            _dma(lax.rem(i - k, N_BUF), i - k).wait()


def _tc_y(ys2, m2, w2, b2):
    G, R = ys2.shape
    return pl.pallas_call(
        _tc_body,
        grid=(G // BLK_MAJ,),
        in_specs=[
            pl.BlockSpec((BLK_MAJ, R), lambda i: (i, 0)),
            pl.BlockSpec((BLK_MAJ, R), lambda i: (i, 0)),
            pl.BlockSpec((1, DIM), lambda i: (0, 0)),
            pl.BlockSpec((1, DIM), lambda i: (0, 0)),
        ],
        out_specs=pl.BlockSpec(memory_space=pl.ANY),
        out_shape=jax.ShapeDtypeStruct((G, R, DIM), jnp.float32),
        scratch_shapes=[
            pltpu.VMEM((N_BUF, BLK_MAJ, R, DIM), jnp.float32),
            pltpu.SemaphoreType.DMA((N_BUF,)),
        ],
    )(ys2, m2, w2, b2)


def _sc_fill(mask_table, batch, n_query):
    info = plsc.get_sparse_core_info()
    nc, ns = info.num_cores, info.num_subcores
    nw = nc * ns
    total_elems = batch * n_query * DIM
    chunk_elems = BUF_ROWS * DIM
    elems_per_w = total_elems // nw
    n_chunk = elems_per_w // chunk_elems
    mesh = plsc.VectorSubcoreMesh(core_axis_name="c", subcore_axis_name="s")

    @functools.partial(
        pl.kernel,
        out_type=jax.ShapeDtypeStruct((total_elems,), jnp.float32),
        mesh=mesh,
        scratch_types=[
            pltpu.VMEM((DIM,), jnp.float32),
            pltpu.VMEM((chunk_elems,), jnp.float32),
            pltpu.SemaphoreType.DMA,
        ],
    )
    def yq_fill(mt_hbm, out_hbm, row_v, buf_v, sem):
        wid = lax.axis_index("s") * nc + lax.axis_index("c")
        base = wid * elems_per_w
        pltpu.sync_copy(mt_hbm, row_v)
        vs = [row_v[pl.ds(LANES * d, LANES)] for d in range(DIM // LANES)]

        def fill_body(i, carry):
            for d in range(DIM // LANES):
                buf_v[pl.ds(i * DIM + LANES * d, LANES)] = vs[d]
            return carry

        lax.fori_loop(0, BUF_ROWS, fill_body, 0)

        def fire_body(j, carry):
            pltpu.make_async_copy(
                buf_v, out_hbm.at[pl.ds(base + j * chunk_elems, chunk_elems)], sem
            ).start()
            return carry

        lax.fori_loop(0, n_chunk, fire_body, 0)

        def drain_body(j, carry):
            pltpu.make_async_copy(
                buf_v, out_hbm.at[pl.ds(base, chunk_elems)], sem
            ).wait()
            return carry

        lax.fori_loop(0, n_chunk, drain_body, 0)

    return yq_fill(mask_table.reshape(DIM))


def kernel(y_support, padding_obs_support, n_obs_query, W_y, b_y, mask_table):
    batch, n_sup = y_support.shape
    total = batch * n_sup
    R = BLK_R
    G = total // R
    ys2 = y_support.reshape(G, R)
    m2 = jnp.where(padding_obs_support.reshape(G, R), 0.0, 1.0).astype(jnp.float32)
    w2 = W_y.reshape(1, DIM)
    b2 = b_y.reshape(1, DIM)

    yq = _sc_fill(mask_table, batch, n_sup)
    y = _tc_y(ys2, m2, w2, b2)

    return (
        y.reshape(batch, n_sup, DIM),
        yq.reshape(batch, n_sup, 1, DIM),
    )
